# baseline (speedup 1.0000x reference)
import functools

import jax
import jax.numpy as jnp
from jax import lax
from jax.experimental import pallas as pl
from jax.experimental.pallas import tpu as pltpu
from jax.experimental.pallas import tpu_sc as plsc

VOCAB = 1000000
DIM = 64
ROWS = 16384 * 50
NUM_WORKERS = 32
PER_W2 = ROWS // 2 // NUM_WORKERS
NBUF = 4
C2 = 160
NCH = PER_W2 // C2
NOUT = NCH // NBUF

_mesh = plsc.VectorSubcoreMesh(core_axis_name="c", subcore_axis_name="s")


@functools.partial(
    pl.kernel,
    mesh=_mesh,
    out_type=jax.ShapeDtypeStruct((ROWS // 2, 2 * DIM), jnp.float32),
    scratch_types=[
        pltpu.VMEM((PER_W2,), jnp.int32),
        pltpu.VMEM((PER_W2,), jnp.int32),
        pltpu.VMEM((NBUF, C2, DIM), jnp.float32),
        pltpu.VMEM((NBUF, C2, DIM), jnp.float32),
        pltpu.SemaphoreType.DMA,
        pltpu.SemaphoreType.DMA,
        pltpu.SemaphoreType.DMA,
        pltpu.SemaphoreType.DMA,
        pltpu.SemaphoreType.DMA,
        pltpu.SemaphoreType.DMA,
        pltpu.SemaphoreType.DMA,
        pltpu.SemaphoreType.DMA,
    ],
    compiler_params=pltpu.CompilerParams(use_tc_tiling_on_sc=False),
)
def _gather(idx_e_hbm, idx_o_hbm, table_hbm, out_hbm, idx_e, idx_o,
            rows_e, rows_o, g0, g1, g2, g3, w0, w1, w2, w3):
    gsem = (g0, g1, g2, g3)
    wsem = (w0, w1, w2, w3)
    wid = lax.axis_index("s") * 2 + lax.axis_index("c")
    base2 = wid * PER_W2
    pltpu.sync_copy(idx_e_hbm.at[pl.ds(base2, PER_W2)], idx_e)
    pltpu.sync_copy(idx_o_hbm.at[pl.ds(base2, PER_W2)], idx_o)

    def in_copies(off2, b):
        return (
            pltpu.make_async_copy(
                table_hbm.at[idx_e.at[pl.ds(off2, C2)]], rows_e.at[b], gsem[b]),
            pltpu.make_async_copy(
                table_hbm.at[idx_o.at[pl.ds(off2, C2)]], rows_o.at[b], gsem[b]),
        )

    def out_copies(off2, b):
        return (
            pltpu.make_async_copy(
                rows_e.at[b], out_hbm.at[pl.ds(base2 + off2, C2), pl.ds(0, DIM)],
                wsem[b]),
            pltpu.make_async_copy(
                rows_o.at[b], out_hbm.at[pl.ds(base2 + off2, C2), pl.ds(DIM, DIM)],
                wsem[b]),
        )

    def in_start(off2, b):
        for c in in_copies(off2, b):
            c.start()

    def in_wait(off2, b):
        for c in in_copies(off2, b):
            c.wait()

    def out_start(off2, b):
        for c in out_copies(off2, b):
            c.start()

    def out_wait(off2, b):
        for c in out_copies(off2, b):
            c.wait()

    for b in range(NBUF):
        in_start(b * C2, b)

    def body(g, carry):
        for b in range(NBUF):
            off2 = pl.multiple_of((g * NBUF + b) * C2, C2)
            in_wait(off2, b)
            out_start(off2, b)
            out_wait(off2, b)
            in_start(off2 + NBUF * C2, b)
        return carry

    lax.fori_loop(0, NOUT - 1, body, 0)

    for b in range(NBUF):
        off2 = ((NOUT - 1) * NBUF + b) * C2
        in_wait(off2, b)
        out_start(off2, b)
    for b in range(NBUF):
        off2 = ((NOUT - 1) * NBUF + b) * C2
        out_wait(off2, b)


def kernel(x, weight):
    idx2 = x.reshape(ROWS // 2, 2)
    out = _gather(idx2[:, 0], idx2[:, 1], weight)
    return out.reshape(16384, 50, DIM)
